# Initial kernel scaffold; baseline (speedup 1.0000x reference)
import jax
import jax.numpy as jnp
from jax.experimental import pallas as pl


def kernel(x, y):
    raise NotImplementedError("write your pallas kernel here")


# speedup vs baseline: 1.4031x; 1.4031x over previous
import functools

import jax
import jax.numpy as jnp
from jax.experimental import pallas as pl
from jax.experimental.pallas import tpu as pltpu

_NT = 256


def _chamfer_tc_body(x_ref, yt_ref, out_ref, colmin_ref, accs_ref, *, nb, b_, n, m):
    i = pl.program_id(1)
    b = pl.program_id(0)

    xb = x_ref[0]
    yt = yt_ref[0]

    acc = jnp.abs(xb[:, 0:1] - yt[0:1, :])
    acc = acc + jnp.abs(xb[:, 1:2] - yt[1:2, :])
    acc = acc + jnp.abs(xb[:, 2:3] - yt[2:3, :])

    @pl.when(jnp.logical_and(b == 0, i == 0))
    def _init():
        accs_ref[0, 0] = 0.0
        accs_ref[0, 1] = 0.0

    @pl.when(i == 0)
    def _reset_colmin():
        colmin_ref[...] = jnp.full_like(colmin_ref, jnp.inf)

    row_min = jnp.min(acc, axis=1)
    accs_ref[0, 0] += jnp.sum(row_min)

    colmin_ref[...] = jnp.minimum(colmin_ref[...], jnp.min(acc, axis=0, keepdims=True))

    @pl.when(i == nb - 1)
    def _finish_batch():
        accs_ref[0, 1] += jnp.sum(colmin_ref[...])

    @pl.when(jnp.logical_and(b == b_ - 1, i == nb - 1))
    def _finish():
        out_ref[0, 0] = accs_ref[0, 0] / (b_ * n) + accs_ref[0, 1] / (b_ * m)


def _chamfer_tc(x, y):
    b_, n, d = x.shape
    m = y.shape[1]
    yt = jnp.transpose(y, (0, 2, 1))
    nb = n // _NT
    out = pl.pallas_call(
        functools.partial(_chamfer_tc_body, nb=nb, b_=b_, n=n, m=m),
        grid=(b_, nb),
        in_specs=[
            pl.BlockSpec((1, _NT, d), lambda b, i: (b, i, 0)),
            pl.BlockSpec((1, d, m), lambda b, i: (b, 0, 0)),
        ],
        out_specs=pl.BlockSpec((1, 1), lambda b, i: (0, 0), memory_space=pltpu.SMEM),
        out_shape=jax.ShapeDtypeStruct((1, 1), jnp.float32),
        scratch_shapes=[
            pltpu.VMEM((1, m), jnp.float32),
            pltpu.SMEM((1, 2), jnp.float32),
        ],
    )(x, yt)
    return out[0, 0]


def kernel(x, y):
    return _chamfer_tc(x, y)
